# baseline (speedup 1.0000x reference)
import functools

import jax
import jax.numpy as jnp
from jax import lax
from jax.experimental import pallas as pl
from jax.experimental.pallas import tpu as pltpu
from jax.experimental.pallas import tpu_sc as plsc

VOCAB_ROWS = 1002
EMBED = 64
B, L = 16384, 50
N_IDX = B * L

_info = plsc.get_sparse_core_info()
NC, NS = _info.num_cores, _info.num_subcores
NW = NC * NS
B_PER_W = N_IDX // NW
CHUNK = 640
T_STEPS = B_PER_W // CHUNK
NBUF = 2


def _gather_body(idx_hbm, table_hbm, out_hbm, idx_v, rows_v,
                 sem_g0, sem_g1, sem_w0, sem_w1):
    wid = lax.axis_index("s") * NC + lax.axis_index("c")
    base = wid * B_PER_W
    sems_g = (sem_g0, sem_g1)
    sems_w = (sem_w0, sem_w1)

    pltpu.sync_copy(idx_hbm.at[pl.ds(base, B_PER_W)], idx_v)

    def start_gather(t, b):
        pltpu.async_copy(
            table_hbm.at[idx_v.at[pl.ds(t * CHUNK, CHUNK)]],
            rows_v.at[b], sems_g[b])

    def wait_gather(b):
        pltpu.make_async_copy(
            table_hbm.at[idx_v.at[pl.ds(0, CHUNK)]],
            rows_v.at[b], sems_g[b]).wait()

    def start_write(t, b):
        pltpu.async_copy(
            rows_v.at[b], out_hbm.at[pl.ds(base + t * CHUNK, CHUNK)],
            sems_w[b])

    def wait_write(b):
        pltpu.make_async_copy(
            rows_v.at[b], out_hbm.at[pl.ds(base, CHUNK)], sems_w[b]).wait()

    for b in range(NBUF):
        start_gather(b, b)

    @pl.loop(0, T_STEPS, step=NBUF)
    def _outer(t0):
        for b in range(NBUF):
            t = t0 + b
            wait_gather(b)

            @pl.when(t >= NBUF)
            def _():
                wait_write(b)

            @pl.when(t + NBUF < T_STEPS)
            def _():
                start_gather(t + NBUF, b)

            start_write(t, b)

    for b in range(NBUF):
        wait_write(b)


@jax.jit
def _embed_lookup(idx_flat, table):
    mesh = plsc.VectorSubcoreMesh(core_axis_name="c", subcore_axis_name="s")
    return pl.kernel(
        _gather_body,
        out_type=jax.ShapeDtypeStruct((N_IDX, EMBED), jnp.float32),
        mesh=mesh,
        scratch_types=[
            pltpu.VMEM((B_PER_W,), jnp.int32),
            pltpu.VMEM((NBUF, CHUNK, EMBED), jnp.float32),
            pltpu.SemaphoreType.DMA,
            pltpu.SemaphoreType.DMA,
            pltpu.SemaphoreType.DMA,
            pltpu.SemaphoreType.DMA,
        ],
        compiler_params=pltpu.CompilerParams(use_tc_tiling_on_sc=False),
    )(idx_flat, table)


def kernel(indices, table):
    out = _embed_lookup(indices.reshape(N_IDX), table)
    return out.reshape(B, L, EMBED)


# speedup vs baseline: 1.0025x; 1.0025x over previous
import functools

import jax
import jax.numpy as jnp
from jax import lax
from jax.experimental import pallas as pl
from jax.experimental.pallas import tpu as pltpu
from jax.experimental.pallas import tpu_sc as plsc

VOCAB_ROWS = 1002
EMBED = 64
B, L = 16384, 50
N_IDX = B * L

_info = plsc.get_sparse_core_info()
NC, NS = _info.num_cores, _info.num_subcores
NW = NC * NS
B_PER_W = N_IDX // NW
CHUNK = 320
T_STEPS = B_PER_W // CHUNK
NBUF = 4


def _gather_body(idx_hbm, table_hbm, out_hbm, idx_v, rows_v,
                 sem_g0, sem_g1, sem_g2, sem_g3,
                 sem_w0, sem_w1, sem_w2, sem_w3):
    wid = lax.axis_index("s") * NC + lax.axis_index("c")
    base = wid * B_PER_W
    sems_g = (sem_g0, sem_g1, sem_g2, sem_g3)
    sems_w = (sem_w0, sem_w1, sem_w2, sem_w3)

    pltpu.sync_copy(idx_hbm.at[pl.ds(base, B_PER_W)], idx_v)

    def start_gather(t, b):
        pltpu.async_copy(
            table_hbm.at[idx_v.at[pl.ds(t * CHUNK, CHUNK)]],
            rows_v.at[b], sems_g[b])

    def wait_gather(b):
        pltpu.make_async_copy(
            table_hbm.at[idx_v.at[pl.ds(0, CHUNK)]],
            rows_v.at[b], sems_g[b]).wait()

    def start_write(t, b):
        pltpu.async_copy(
            rows_v.at[b], out_hbm.at[pl.ds(base + t * CHUNK, CHUNK)],
            sems_w[b])

    def wait_write(b):
        pltpu.make_async_copy(
            rows_v.at[b], out_hbm.at[pl.ds(base, CHUNK)], sems_w[b]).wait()

    for b in range(NBUF):
        start_gather(b, b)

    @pl.loop(0, T_STEPS, step=NBUF)
    def _outer(t0):
        for b in range(NBUF):
            t = t0 + b
            wait_gather(b)
            start_write(t, b)
            bp = (b - 1) % NBUF
            tp = t - 1 + NBUF

            @pl.when(jnp.logical_and(t >= 1, tp < T_STEPS))
            def _():
                wait_write(bp)
                start_gather(tp, bp)

    for b in range(NBUF):
        wait_write(b)


@jax.jit
def _embed_lookup(idx_flat, table):
    mesh = plsc.VectorSubcoreMesh(core_axis_name="c", subcore_axis_name="s")
    return pl.kernel(
        _gather_body,
        out_type=jax.ShapeDtypeStruct((N_IDX, EMBED), jnp.float32),
        mesh=mesh,
        scratch_types=[
            pltpu.VMEM((B_PER_W,), jnp.int32),
            pltpu.VMEM((NBUF, CHUNK, EMBED), jnp.float32),
            pltpu.SemaphoreType.DMA,
            pltpu.SemaphoreType.DMA,
            pltpu.SemaphoreType.DMA,
            pltpu.SemaphoreType.DMA,
            pltpu.SemaphoreType.DMA,
            pltpu.SemaphoreType.DMA,
            pltpu.SemaphoreType.DMA,
            pltpu.SemaphoreType.DMA,
        ],
        compiler_params=pltpu.CompilerParams(use_tc_tiling_on_sc=False),
    )(idx_flat, table)


def kernel(indices, table):
    out = _embed_lookup(indices.reshape(N_IDX), table)
    return out.reshape(B, L, EMBED)


# speedup vs baseline: 1.0042x; 1.0017x over previous
import functools

import jax
import jax.numpy as jnp
from jax import lax
from jax.experimental import pallas as pl
from jax.experimental.pallas import tpu as pltpu
from jax.experimental.pallas import tpu_sc as plsc

VOCAB_ROWS = 1002
EMBED = 64
B, L = 16384, 50
N_IDX = B * L

_info = plsc.get_sparse_core_info()
NC, NS = _info.num_cores, _info.num_subcores
NW = NC * NS
B_PER_W = N_IDX // NW
CHUNK = 400
ROWS_PER_CHUNK = CHUNK // L
T_STEPS = B_PER_W // CHUNK
NBUF = 4
assert B_PER_W % CHUNK == 0 and T_STEPS % NBUF == 0 and CHUNK % L == 0


def _gather_body(idx_hbm, table_hbm, out_hbm3d, idx_v, rows_v,
                 sem_g0, sem_g1, sem_g2, sem_g3,
                 sem_w0, sem_w1, sem_w2, sem_w3):
    wid = lax.axis_index("s") * NC + lax.axis_index("c")
    base = wid * B_PER_W
    row_base = wid * (B_PER_W // L)
    sems_g = (sem_g0, sem_g1, sem_g2, sem_g3)
    sems_w = (sem_w0, sem_w1, sem_w2, sem_w3)

    pltpu.sync_copy(idx_hbm.at[pl.ds(base, B_PER_W)], idx_v)

    def start_gather(t, b):
        pltpu.async_copy(
            table_hbm.at[idx_v.at[pl.ds(t * CHUNK, CHUNK)]],
            rows_v.at[b], sems_g[b])

    def wait_gather(b):
        pltpu.make_async_copy(
            table_hbm.at[idx_v.at[pl.ds(0, CHUNK)]],
            rows_v.at[b], sems_g[b]).wait()

    def start_write(t, b):
        for r in range(ROWS_PER_CHUNK):
            pltpu.async_copy(
                rows_v.at[b].at[pl.ds(r * L, L)],
                out_hbm3d.at[row_base + t * ROWS_PER_CHUNK + r],
                sems_w[b])

    def wait_write(b):
        for r in range(ROWS_PER_CHUNK):
            pltpu.make_async_copy(
                rows_v.at[b].at[pl.ds(r * L, L)],
                out_hbm3d.at[row_base], sems_w[b]).wait()

    for b in range(NBUF):
        start_gather(b, b)

    @pl.loop(0, T_STEPS, step=NBUF)
    def _outer(t0):
        for b in range(NBUF):
            t = t0 + b
            wait_gather(b)
            start_write(t, b)
            bp = (b - 1) % NBUF
            tp = t - 1 + NBUF

            @pl.when(jnp.logical_and(t >= 1, tp < T_STEPS))
            def _():
                wait_write(bp)
                start_gather(tp, bp)

    for b in range(NBUF):
        wait_write(b)


@jax.jit
def _embed_lookup(idx_flat, table):
    mesh = plsc.VectorSubcoreMesh(core_axis_name="c", subcore_axis_name="s")
    return pl.kernel(
        _gather_body,
        out_type=jax.ShapeDtypeStruct((B, L, EMBED), jnp.float32),
        mesh=mesh,
        scratch_types=[
            pltpu.VMEM((B_PER_W,), jnp.int32),
            pltpu.VMEM((NBUF, CHUNK, EMBED), jnp.float32),
            pltpu.SemaphoreType.DMA,
            pltpu.SemaphoreType.DMA,
            pltpu.SemaphoreType.DMA,
            pltpu.SemaphoreType.DMA,
            pltpu.SemaphoreType.DMA,
            pltpu.SemaphoreType.DMA,
            pltpu.SemaphoreType.DMA,
            pltpu.SemaphoreType.DMA,
        ],
        compiler_params=pltpu.CompilerParams(use_tc_tiling_on_sc=False),
    )(idx_flat, table)


def kernel(indices, table):
    return _embed_lookup(indices.reshape(N_IDX), table)
